# Initial kernel scaffold; baseline (speedup 1.0000x reference)
import jax
import jax.numpy as jnp
from jax import lax
from jax.experimental import pallas as pl
from jax.experimental.pallas import tpu as pltpu
from jax.experimental.pallas import tpu_sc as plsc

B = 16384
DIM = 64
NUM_CORES = 2
NUM_SUBCORES = 16
NW = NUM_CORES * NUM_SUBCORES
BPW = B // NW


def _body(rt_ref, ln_ref, tp_ref, wr_ref, wl_ref, wt_ref, out_ref,
          idx0, idx1, idx2, rows0, rows1, rows2, sem):
    wid = lax.axis_index("s") * NUM_CORES + lax.axis_index("c")
    base = wid * BPW

    pltpu.sync_copy(rt_ref.at[pl.ds(base, BPW)], idx0)
    pltpu.sync_copy(ln_ref.at[pl.ds(base, BPW)], idx1)
    pltpu.sync_copy(tp_ref.at[pl.ds(base, BPW)], idx2)

    c0 = pltpu.async_copy(wr_ref.at[idx0], rows0, sem)
    c1 = pltpu.async_copy(wl_ref.at[idx1], rows1, sem)
    c2 = pltpu.async_copy(wt_ref.at[idx2], rows2, sem)
    c0.wait()
    c1.wait()
    c2.wait()

    pltpu.sync_copy(rows0, out_ref.at[pl.ds(base, BPW), pl.ds(0, DIM)])
    pltpu.sync_copy(rows1, out_ref.at[pl.ds(base, BPW), pl.ds(DIM, DIM)])
    pltpu.sync_copy(rows2, out_ref.at[pl.ds(base, BPW), pl.ds(2 * DIM, DIM)])


@jax.jit
def _lookup_concat(road_type, lane, time_period, W_road, W_lane, W_time):
    mesh = plsc.VectorSubcoreMesh(core_axis_name="c", subcore_axis_name="s")
    return pl.kernel(
        _body,
        out_type=jax.ShapeDtypeStruct((B, 3 * DIM), jnp.float32),
        mesh=mesh,
        scratch_types=[
            pltpu.VMEM((BPW,), jnp.int32),
            pltpu.VMEM((BPW,), jnp.int32),
            pltpu.VMEM((BPW,), jnp.int32),
            pltpu.VMEM((BPW, DIM), jnp.float32),
            pltpu.VMEM((BPW, DIM), jnp.float32),
            pltpu.VMEM((BPW, DIM), jnp.float32),
            pltpu.SemaphoreType.DMA,
        ],
    )(road_type, lane, time_period, W_road, W_lane, W_time)


def kernel(road_type, lane, time_period, W_road, W_lane, W_time):
    return _lookup_concat(
        road_type.astype(jnp.int32),
        lane.astype(jnp.int32),
        time_period.astype(jnp.int32),
        W_road, W_lane, W_time,
    )


# speedup vs baseline: 3.4202x; 3.4202x over previous
import jax
import jax.numpy as jnp
from jax import lax
from jax.experimental import pallas as pl
from jax.experimental.pallas import tpu as pltpu
from jax.experimental.pallas import tpu_sc as plsc

B = 16384
DIM = 64
NUM_CORES = 2
NUM_SUBCORES = 16
NW = NUM_CORES * NUM_SUBCORES
BPW = B // NW


def _body(rt_ref, ln_ref, tp_ref, wr_ref, wl_ref, wt_ref, out_ref,
          idx0, idx1, idx2, rows0, rows1, rows2, sem):
    wid = lax.axis_index("s") * NUM_CORES + lax.axis_index("c")
    base = wid * BPW

    pltpu.sync_copy(rt_ref.at[pl.ds(base, BPW)], idx0)
    pltpu.sync_copy(ln_ref.at[pl.ds(base, BPW)], idx1)
    pltpu.sync_copy(tp_ref.at[pl.ds(base, BPW)], idx2)

    c0 = pltpu.async_copy(wr_ref.at[idx0], rows0, sem)
    c1 = pltpu.async_copy(wl_ref.at[idx1], rows1, sem)
    c2 = pltpu.async_copy(wt_ref.at[idx2], rows2, sem)
    c0.wait()
    c1.wait()
    c2.wait()

    pltpu.sync_copy(rows0, out_ref.at[pl.ds(base, BPW), pl.ds(0, DIM)])
    pltpu.sync_copy(rows1, out_ref.at[pl.ds(base, BPW), pl.ds(DIM, DIM)])
    pltpu.sync_copy(rows2, out_ref.at[pl.ds(base, BPW), pl.ds(2 * DIM, DIM)])


@jax.jit
def _lookup_concat(road_type, lane, time_period, W_road, W_lane, W_time):
    mesh = plsc.VectorSubcoreMesh(core_axis_name="c", subcore_axis_name="s")
    return pl.kernel(
        _body,
        out_type=jax.ShapeDtypeStruct((B, 3 * DIM), jnp.float32),
        mesh=mesh,
        compiler_params=pltpu.CompilerParams(use_tc_tiling_on_sc=False),
        scratch_types=[
            pltpu.VMEM((BPW,), jnp.int32),
            pltpu.VMEM((BPW,), jnp.int32),
            pltpu.VMEM((BPW,), jnp.int32),
            pltpu.VMEM((BPW, DIM), jnp.float32),
            pltpu.VMEM((BPW, DIM), jnp.float32),
            pltpu.VMEM((BPW, DIM), jnp.float32),
            pltpu.SemaphoreType.DMA,
        ],
    )(road_type, lane, time_period, W_road, W_lane, W_time)


def kernel(road_type, lane, time_period, W_road, W_lane, W_time):
    return _lookup_concat(
        road_type.astype(jnp.int32),
        lane.astype(jnp.int32),
        time_period.astype(jnp.int32),
        W_road, W_lane, W_time,
    )
